# Initial kernel scaffold; baseline (speedup 1.0000x reference)
import jax
import jax.numpy as jnp
from jax.experimental import pallas as pl


def kernel(context_features, context_labels):
    raise NotImplementedError("write your pallas kernel here")


# speedup vs baseline: 5.9295x; 5.9295x over previous
import functools

import jax
import jax.numpy as jnp
from jax import lax
from jax.experimental import pallas as pl
from jax.experimental.pallas import tpu as pltpu
from jax.experimental.pallas import tpu_sc as plsc

_NUM_CLASSES = 1000
_C_PAD = 1024
_N = 320000
_D = 128
_CHUNK = 128
_NCHUNKS = _N // _CHUNK
_NC = 2
_NS = 16
_NW = _NC * _NS
_CHUNKS_PER_W = -(-_NCHUNKS // _NW)
_ROWS_PER_TILE = _C_PAD // _NS

_mesh = plsc.VectorSubcoreMesh(core_axis_name="c", subcore_axis_name="s")


@functools.partial(
    pl.kernel,
    out_type=(
        jax.ShapeDtypeStruct((_NC, _C_PAD, _D), jnp.float32),
        jax.ShapeDtypeStruct((_NW, _C_PAD), jnp.float32),
    ),
    mesh=_mesh,
    compiler_params=pltpu.CompilerParams(needs_layout_passes=False),
    scratch_types=[
        pltpu.VMEM((_CHUNK, _D), jnp.float32),
        pltpu.VMEM((_CHUNK,), jnp.int32),
        pltpu.VMEM((_C_PAD,), jnp.float32),
        pltpu.VMEM_SHARED((_C_PAD, _D), jnp.float32),
    ],
)
def _segment_sums(feat, lab2d, zsum, psums, pcnts,
                  rows_v, idx_v, cnt_v, acc_s):
    c = lax.axis_index("c")
    s = lax.axis_index("s")
    w = s * _NC + c

    base = s * _ROWS_PER_TILE
    pltpu.sync_copy(zsum.at[pl.ds(base, _ROWS_PER_TILE)],
                    acc_s.at[pl.ds(base, _ROWS_PER_TILE)])

    zv = jnp.zeros((16,), jnp.float32)

    def zbody(k, carry):
        cnt_v[pl.ds(k * 16, 16)] = zv
        return carry

    lax.fori_loop(0, _C_PAD // 16, zbody, 0)
    plsc.subcore_barrier()

    ones16 = jnp.ones((16,), jnp.float32)

    def body(i, carry):
        j = w * _CHUNKS_PER_W + i

        @pl.when(j < _NCHUNKS)
        def _():
            pltpu.sync_copy(feat.at[pl.ds(j * _CHUNK, _CHUNK)], rows_v)
            pltpu.sync_copy(lab2d.at[j], idx_v)
            pltpu.sync_copy(rows_v, acc_s.at[idx_v], add=True)
            for k in range(_CHUNK // 16):
                idx = idx_v[pl.ds(k * 16, 16)]
                plsc.addupdate_scatter(cnt_v, [idx], ones16)

        return carry

    lax.fori_loop(0, _CHUNKS_PER_W, body, 0)

    plsc.subcore_barrier()
    pltpu.sync_copy(acc_s.at[pl.ds(base, _ROWS_PER_TILE)],
                    psums.at[c, pl.ds(base, _ROWS_PER_TILE)])
    pltpu.sync_copy(cnt_v, pcnts.at[w])


def _combine_body(ps_ref, pc_ref, o_ref):
    sums = ps_ref[0] + ps_ref[1]
    cnts = jnp.sum(pc_ref[...], axis=0)
    denom = jnp.maximum(cnts[:, None], 1.0)
    o_ref[...] = (sums / denom)[:_NUM_CLASSES]


def kernel(context_features, context_labels):
    labels = context_labels.astype(jnp.int32).reshape(_NCHUNKS, _CHUNK)
    zsum = jnp.zeros((_C_PAD, _D), jnp.float32)
    psums, pcnts = _segment_sums(context_features, labels, zsum)
    return pl.pallas_call(
        _combine_body,
        out_shape=jax.ShapeDtypeStruct((_NUM_CLASSES, _D), jnp.float32),
    )(psums, pcnts)


# speedup vs baseline: 9.3085x; 1.5698x over previous
import functools

import jax
import jax.numpy as jnp
from jax import lax
from jax.experimental import pallas as pl
from jax.experimental.pallas import tpu as pltpu
from jax.experimental.pallas import tpu_sc as plsc

_NUM_CLASSES = 1000
_C_PAD = 1024
_N = 320000
_D = 128
_CHUNK = 256
_SUB = 128
_NSUB = _CHUNK // _SUB
_NCHUNKS = _N // _CHUNK
_NC = 2
_NS = 16
_NW = _NC * _NS
_CHUNKS_PER_W = -(-_NCHUNKS // _NW)
_ROWS_PER_TILE = _C_PAD // _NS

_mesh = plsc.VectorSubcoreMesh(core_axis_name="c", subcore_axis_name="s")


@functools.partial(
    pl.kernel,
    out_type=(
        jax.ShapeDtypeStruct((_NC, _C_PAD, _D), jnp.float32),
        jax.ShapeDtypeStruct((_NW, _C_PAD), jnp.float32),
    ),
    mesh=_mesh,
    compiler_params=pltpu.CompilerParams(needs_layout_passes=False),
    scratch_types=[
        pltpu.VMEM((_CHUNK, _D), jnp.float32),
        pltpu.VMEM((_CHUNK, _D), jnp.float32),
        pltpu.VMEM((_SUB,), jnp.int32),
        pltpu.VMEM((_SUB,), jnp.int32),
        pltpu.VMEM((_SUB,), jnp.int32),
        pltpu.VMEM((_SUB,), jnp.int32),
        pltpu.VMEM((_C_PAD,), jnp.float32),
        pltpu.VMEM_SHARED((_C_PAD, _D), jnp.float32),
        pltpu.SemaphoreType.DMA,
        pltpu.SemaphoreType.DMA,
        pltpu.SemaphoreType.DMA,
        pltpu.SemaphoreType.DMA,
    ],
)
def _segment_sums(feat, lab1d, zsum, psums, pcnts,
                  rows0, rows1, idx00, idx01, idx10, idx11, cnt_v, acc_s,
                  lsem0, lsem1, isem0, isem1):
    rows = (rows0, rows1)
    idx = ((idx00, idx01), (idx10, idx11))
    lsem = (lsem0, lsem1)
    isem = (isem0, isem1)

    c = lax.axis_index("c")
    s = lax.axis_index("s")
    w = s * _NC + c
    jbase = w * _CHUNKS_PER_W
    jend = jnp.minimum(jbase + _CHUNKS_PER_W, _NCHUNKS)

    base = s * _ROWS_PER_TILE
    pltpu.sync_copy(zsum.at[pl.ds(base, _ROWS_PER_TILE)],
                    acc_s.at[pl.ds(base, _ROWS_PER_TILE)])

    zv = jnp.zeros((16,), jnp.float32)

    def zbody(k, carry):
        cnt_v[pl.ds(k * 16, 16)] = zv
        return carry

    lax.fori_loop(0, _C_PAD // 16, zbody, 0)
    plsc.subcore_barrier()

    ones16 = jnp.ones((16,), jnp.float32)

    def start_load(b, j):
        pltpu.async_copy(feat.at[pl.ds(j * _CHUNK, _CHUNK)], rows[b], lsem[b])
        for k in range(_NSUB):
            pltpu.async_copy(lab1d.at[pl.ds(j * _CHUNK + k * _SUB, _SUB)],
                             idx[b][k], isem[b])

    def wait_load(b, j):
        pltpu.make_async_copy(feat.at[pl.ds(j * _CHUNK, _CHUNK)],
                              rows[b], lsem[b]).wait()
        for k in range(_NSUB):
            pltpu.make_async_copy(
                lab1d.at[pl.ds(j * _CHUNK + k * _SUB, _SUB)],
                idx[b][k], isem[b]).wait()

    for b in range(2):
        jj = jbase + b

        @pl.when(jj < jend)
        def _():
            start_load(b, jj)

    def body(i, carry):
        for b in range(2):
            j = jbase + 2 * i + b

            @pl.when(j < jend)
            def _():
                wait_load(b, j)
                for k in range(_NSUB):
                    pltpu.sync_copy(rows[b].at[pl.ds(k * _SUB, _SUB)],
                                    acc_s.at[idx[b][k]], add=True)
                for k in range(_NSUB):
                    for m in range(_SUB // 16):
                        iv = idx[b][k][pl.ds(m * 16, 16)]
                        plsc.addupdate_scatter(cnt_v, [iv], ones16)
                jn = j + 2

                @pl.when(jn < jend)
                def _():
                    start_load(b, jn)

        return carry

    lax.fori_loop(0, _CHUNKS_PER_W // 2, body, 0)

    plsc.subcore_barrier()
    pltpu.sync_copy(acc_s.at[pl.ds(base, _ROWS_PER_TILE)],
                    psums.at[c, pl.ds(base, _ROWS_PER_TILE)])
    pltpu.sync_copy(cnt_v, pcnts.at[w])


def _combine_body(ps_ref, pc_ref, o_ref):
    sums = ps_ref[0] + ps_ref[1]
    cnts = jnp.sum(pc_ref[...], axis=0)
    denom = jnp.maximum(cnts[:, None], 1.0)
    o_ref[...] = (sums / denom)[:_NUM_CLASSES]


def kernel(context_features, context_labels):
    labels = context_labels.astype(jnp.int32)
    zsum = jnp.zeros((_C_PAD, _D), jnp.float32)
    psums, pcnts = _segment_sums(context_features, labels, zsum)
    return pl.pallas_call(
        _combine_body,
        out_shape=jax.ShapeDtypeStruct((_NUM_CLASSES, _D), jnp.float32),
    )(psums, pcnts)


# speedup vs baseline: 11.4106x; 1.2258x over previous
import functools

import jax
import jax.numpy as jnp
from jax import lax
from jax.experimental import pallas as pl
from jax.experimental.pallas import tpu as pltpu
from jax.experimental.pallas import tpu_sc as plsc

_NUM_CLASSES = 1000
_C_PAD = 1024
_N = 320000
_D = 128
_CHUNK = 256
_SUB = 128
_NSUB = _CHUNK // _SUB
_NCHUNKS = _N // _CHUNK
_NC = 2
_NS = 16
_NW = _NC * _NS
_CPW = _NCHUNKS // _NW
_MAX_CPW = _CPW + 1
_NROUNDS = -(-_MAX_CPW // 3)
_ROWS_PER_TILE = _C_PAD // _NS

_mesh = plsc.VectorSubcoreMesh(core_axis_name="c", subcore_axis_name="s")


@functools.partial(
    pl.kernel,
    out_type=(
        jax.ShapeDtypeStruct((_NC, _C_PAD, _D), jnp.float32),
        jax.ShapeDtypeStruct((_NW, _C_PAD), jnp.float32),
    ),
    mesh=_mesh,
    compiler_params=pltpu.CompilerParams(needs_layout_passes=False),
    scratch_types=[
        pltpu.VMEM((_CHUNK, _D), jnp.float32),
        pltpu.VMEM((_CHUNK, _D), jnp.float32),
        pltpu.VMEM((_CHUNK, _D), jnp.float32),
        pltpu.VMEM((_SUB,), jnp.int32),
        pltpu.VMEM((_SUB,), jnp.int32),
        pltpu.VMEM((_SUB,), jnp.int32),
        pltpu.VMEM((_SUB,), jnp.int32),
        pltpu.VMEM((_SUB,), jnp.int32),
        pltpu.VMEM((_SUB,), jnp.int32),
        pltpu.VMEM((_C_PAD,), jnp.float32),
        pltpu.VMEM_SHARED((_C_PAD, _D), jnp.float32),
        pltpu.SemaphoreType.DMA,
        pltpu.SemaphoreType.DMA,
        pltpu.SemaphoreType.DMA,
        pltpu.SemaphoreType.DMA,
        pltpu.SemaphoreType.DMA,
        pltpu.SemaphoreType.DMA,
    ],
)
def _segment_sums(feat, lab1d, zsum, psums, pcnts,
                  rows0, rows1, rows2,
                  idx00, idx01, idx10, idx11, idx20, idx21,
                  cnt_v, acc_s,
                  lsem0, lsem1, lsem2, ssem0, ssem1, ssem2):
    rows = (rows0, rows1, rows2)
    idx = ((idx00, idx01), (idx10, idx11), (idx20, idx21))
    lsem = (lsem0, lsem1, lsem2)
    ssem = (ssem0, ssem1, ssem2)

    c = lax.axis_index("c")
    s = lax.axis_index("s")
    w = s * _NC + c
    jbase = _CPW * w + jnp.minimum(w, 2)
    jend = _CPW * (w + 1) + jnp.minimum(w + 1, 2)
    nch = jend - jbase

    base = s * _ROWS_PER_TILE
    pltpu.sync_copy(zsum.at[pl.ds(base, _ROWS_PER_TILE)],
                    acc_s.at[pl.ds(base, _ROWS_PER_TILE)])

    zv = jnp.zeros((16,), jnp.float32)

    def zbody(k, carry):
        cnt_v[pl.ds(k * 16, 16)] = zv
        return carry

    lax.fori_loop(0, _C_PAD // 16, zbody, 0)
    plsc.subcore_barrier()

    ones16 = jnp.ones((16,), jnp.float32)

    def start_load(r, j):
        pltpu.async_copy(feat.at[pl.ds(j * _CHUNK, _CHUNK)], rows[r], lsem[r])
        for k in range(_NSUB):
            pltpu.async_copy(lab1d.at[pl.ds(j * _CHUNK + k * _SUB, _SUB)],
                             idx[r][k], lsem[r])

    def wait_load(r, j):
        pltpu.make_async_copy(feat.at[pl.ds(j * _CHUNK, _CHUNK)],
                              rows[r], lsem[r]).wait()
        for k in range(_NSUB):
            pltpu.make_async_copy(
                lab1d.at[pl.ds(j * _CHUNK + k * _SUB, _SUB)],
                idx[r][k], lsem[r]).wait()

    def start_scatter(r):
        for k in range(_NSUB):
            pltpu.async_copy(rows[r].at[pl.ds(k * _SUB, _SUB)],
                             acc_s.at[idx[r][k]], ssem[r], add=True)

    def wait_scatter(r):
        for k in range(_NSUB):
            pltpu.make_async_copy(rows[r].at[pl.ds(k * _SUB, _SUB)],
                                  acc_s.at[idx[r][k]], ssem[r]).wait()

    def do_counts(r):
        for k in range(_NSUB):
            for m in range(_SUB // 16):
                iv = idx[r][k][pl.ds(m * 16, 16)]
                plsc.addupdate_scatter(cnt_v, [iv], ones16)

    def step(j, r, has_prev):
        @pl.when(j < jend)
        def _():
            wait_load(r, j)
            start_scatter(r)
            do_counts(r)
            rp = (r - 1) % 3
            if has_prev:
                wait_scatter(rp)
            jn = j + 2

            @pl.when(jn < jend)
            def _():
                start_load(rp, jn)

    for r in range(2):
        jj = jbase + r

        @pl.when(jj < jend)
        def _():
            start_load(r, jj)

    for r in range(3):
        step(jbase + r, r, has_prev=(r != 0))

    def body(i, carry):
        for r in range(3):
            step(jbase + 3 * i + r, r, has_prev=True)
        return carry

    lax.fori_loop(1, _NROUNDS, body, 0)

    for r in range(3):
        @pl.when(lax.rem(nch - 1, 3) == r)
        def _():
            wait_scatter(r)

    plsc.subcore_barrier()
    pltpu.sync_copy(acc_s.at[pl.ds(base, _ROWS_PER_TILE)],
                    psums.at[c, pl.ds(base, _ROWS_PER_TILE)])
    pltpu.sync_copy(cnt_v, pcnts.at[w])


def _combine_body(ps_ref, pc_ref, o_ref):
    sums = ps_ref[0] + ps_ref[1]
    cnts = jnp.sum(pc_ref[...], axis=0)
    denom = jnp.maximum(cnts[:, None], 1.0)
    o_ref[...] = (sums / denom)[:_NUM_CLASSES]


def kernel(context_features, context_labels):
    labels = context_labels.astype(jnp.int32)
    zsum = jnp.zeros((_C_PAD, _D), jnp.float32)
    psums, pcnts = _segment_sums(context_features, labels, zsum)
    return pl.pallas_call(
        _combine_body,
        out_shape=jax.ShapeDtypeStruct((_NUM_CLASSES, _D), jnp.float32),
    )(psums, pcnts)
